# baseline (speedup 1.0000x reference)
import functools

import jax
import jax.numpy as jnp
from jax.experimental import pallas as pl
from jax.experimental.pallas import tpu as pltpu

_N = 50000
_D = 128
_M = 12500
_R = 0.1
_K = 16

_ROWS = 8
_IMAXV = 2**31 - 1


def _fps_body(m, rows, w, wp, planes_ref, lin_ref, pos0_ref, sel_ref,
              qx_ref, qy_ref, qz_ref, dists_ref):
    lin = lin_ref[...]
    dists_ref[...] = jnp.where(lin < jnp.int32(rows * w),
                               jnp.inf, -jnp.inf).astype(jnp.float32)
    lane128 = jax.lax.broadcasted_iota(jnp.int32, (1, 128), 1)
    lanew = jax.lax.broadcasted_iota(jnp.int32, (1, wp), 1)

    def body(i, carry):
        nxt, wx, wy, wz, a_s, a_x, a_y, a_z = carry
        laneq = lane128 == (i % 128)
        a_s = jnp.where(laneq, nxt, a_s)
        a_x = jnp.where(laneq, wx, a_x)
        a_y = jnp.where(laneq, wy, a_y)
        a_z = jnp.where(laneq, wz, a_z)
        blk = i // 128

        @pl.when((i % 128 == 127) | (i == m - 1))
        def _():
            sel_ref[pl.ds(blk, 1), :] = a_s
            qx_ref[pl.ds(blk, 1), :] = a_x
            qy_ref[pl.ds(blk, 1), :] = a_y
            qz_ref[pl.ds(blk, 1), :] = a_z

        dx = planes_ref[0] - wx
        dy = planes_ref[1] - wy
        dz = planes_ref[2] - wz
        d = (dx * dx + dz * dz) + dy * dy
        nd = jnp.minimum(dists_ref[...], d)
        dists_ref[...] = nd
        mx = jnp.max(nd)
        nxt2 = jnp.min(jnp.where(nd == mx, lin, jnp.int32(_IMAXV)))
        r = nxt2 // w
        c = nxt2 % w
        lmask = lanew == c
        wx2 = jnp.sum(jnp.where(lmask, planes_ref[0, pl.ds(r, 1), :], 0.0))
        wy2 = jnp.sum(jnp.where(lmask, planes_ref[1, pl.ds(r, 1), :], 0.0))
        wz2 = jnp.sum(jnp.where(lmask, planes_ref[2, pl.ds(r, 1), :], 0.0))
        return (nxt2, wx2, wy2, wz2, a_s, a_x, a_y, a_z)

    zf = jnp.zeros((1, 128), jnp.float32)
    zi = jnp.zeros((1, 128), jnp.int32)
    init = (jnp.int32(0), pos0_ref[0, 0], pos0_ref[0, 1], pos0_ref[0, 2],
            zi, zf, zf, zf)
    jax.lax.fori_loop(0, m, body, init)


def _fps_pallas(pos, m, interpret=False):
    n = pos.shape[0]
    rows = _ROWS
    w = n // rows
    wp = ((w + 127) // 128) * 128
    nb = (m + 127) // 128
    planes = pos.T.reshape(3, rows, w)
    if wp > w:
        planes = jnp.pad(planes, ((0, 0), (0, 0), (0, wp - w)))
    r_iota = jax.lax.broadcasted_iota(jnp.int32, (rows, wp), 0)
    c_iota = jax.lax.broadcasted_iota(jnp.int32, (rows, wp), 1)
    lin = jnp.where(c_iota < w, r_iota * w + c_iota, jnp.int32(_IMAXV))
    pos0 = pos[0:1, :]

    out = pl.pallas_call(
        functools.partial(_fps_body, m, rows, w, wp),
        grid=(1,),
        in_specs=[
            pl.BlockSpec((3, rows, wp), lambda i: (0, 0, 0)),
            pl.BlockSpec((rows, wp), lambda i: (0, 0)),
            pl.BlockSpec(memory_space=pltpu.SMEM),
        ],
        out_specs=[
            pl.BlockSpec((nb, 128), lambda i: (0, 0)),
            pl.BlockSpec((nb, 128), lambda i: (0, 0)),
            pl.BlockSpec((nb, 128), lambda i: (0, 0)),
            pl.BlockSpec((nb, 128), lambda i: (0, 0)),
        ],
        out_shape=[
            jax.ShapeDtypeStruct((nb, 128), jnp.int32),
            jax.ShapeDtypeStruct((nb, 128), jnp.float32),
            jax.ShapeDtypeStruct((nb, 128), jnp.float32),
            jax.ShapeDtypeStruct((nb, 128), jnp.float32),
        ],
        scratch_shapes=[pltpu.VMEM((rows, wp), jnp.float32)],
        interpret=interpret,
    )(planes, lin, pos0)
    sel = out[0].reshape(-1)[:m]
    pos_q = jnp.stack([o.reshape(-1)[:m] for o in out[1:]], axis=-1)
    return sel, pos_q


def _knn_body(nch, nchp, rr, k, px_ref, py_ref, pz_ref, q_ref,
              nbr_ref, msk_ref, neg_ref, mc_ref):
    qx = q_ref[0, 0]
    qy = q_ref[1, 0]
    qz = q_ref[2, 0]
    mc_ref[...] = jnp.full((8, nchp), -jnp.inf, jnp.float32)
    for j in range(nch):
        dx = px_ref[pl.ds(j, 1), :] - qx
        dy = py_ref[pl.ds(j, 1), :] - qy
        dz = pz_ref[pl.ds(j, 1), :] - qz
        d = (dx * dx + dz * dz) + dy * dy
        ng = jnp.where(d <= rr, -d, -jnp.inf)
        neg_ref[j] = ng
        mc_ref[:, j:j + 1] = jnp.max(ng, axis=1, keepdims=True)

    lane128 = jax.lax.broadcasted_iota(jnp.int32, (1, 128), 1)
    chunk2d = jax.lax.broadcasted_iota(jnp.int32, (8, nchp), 1)
    chunk1d = jax.lax.broadcasted_iota(jnp.int32, (1, nchp), 1)
    sub16 = jax.lax.broadcasted_iota(jnp.int32, (8, k), 0)
    lane16 = jax.lax.broadcasted_iota(jnp.int32, (8, k), 1)
    acc_nbr = jnp.zeros((8, k), jnp.int32)
    acc_msk = jnp.zeros((8, k), jnp.int32)
    for kk in range(k):
        mc = mc_ref[...]
        m = jnp.max(mc, axis=1, keepdims=True)
        jsel = jnp.min(jnp.where(mc == m, chunk2d, jnp.int32(_IMAXV)),
                       axis=1, keepdims=True)
        for q in range(8):
            c = jsel[q, 0]
            mq = m[q, 0]
            row = neg_ref[pl.ds(c, 1), q, :]
            lane = jnp.min(jnp.where(row == mq, lane128, jnp.int32(_IMAXV)))
            n = c * 128 + lane
            nrow = jnp.where(lane128 == lane, -jnp.inf, row)
            neg_ref[pl.ds(c, 1), q, :] = nrow
            nmx = jnp.max(nrow)
            mc_ref[q:q + 1, :] = jnp.where(chunk1d == c, nmx,
                                           mc_ref[q:q + 1, :])
            hit = (sub16 == q) & (lane16 == kk)
            acc_nbr = jnp.where(hit, n, acc_nbr)
            acc_msk = jnp.where(hit, jnp.where(mq > -jnp.inf, 1, 0), acc_msk)
    nbr_ref[...] = acc_nbr
    msk_ref[...] = acc_msk


def _knn_pallas(pos, pos_q, rr, k, interpret=False):
    n = pos.shape[0]
    m = pos_q.shape[0]
    nch = (n + 127) // 128
    nchp = ((nch + 127) // 128) * 128
    nblk = (m + 7) // 8
    mp = nblk * 8
    planes = pos.T
    if nch * 128 > n:
        planes = jnp.pad(planes, ((0, 0), (0, nch * 128 - n)),
                         constant_values=1e9)
    planes = planes.reshape(3, nch, 128)
    qp = pos_q
    if mp > m:
        qp = jnp.pad(qp, ((0, mp - m), (0, 0)), constant_values=2.0)
    qb = qp.T.reshape(3, nblk, 8, 1)

    nbr, msk = pl.pallas_call(
        functools.partial(_knn_body, nch, nchp, rr, k),
        grid=(nblk,),
        in_specs=[
            pl.BlockSpec((nch, 128), lambda b: (0, 0)),
            pl.BlockSpec((nch, 128), lambda b: (0, 0)),
            pl.BlockSpec((nch, 128), lambda b: (0, 0)),
            pl.BlockSpec((3, 1, 8, 1), lambda b: (0, b, 0, 0)),
        ],
        out_specs=[
            pl.BlockSpec((8, k), lambda b: (b, 0)),
            pl.BlockSpec((8, k), lambda b: (b, 0)),
        ],
        out_shape=[
            jax.ShapeDtypeStruct((mp, k), jnp.int32),
            jax.ShapeDtypeStruct((mp, k), jnp.int32),
        ],
        scratch_shapes=[
            pltpu.VMEM((nch, 8, 128), jnp.float32),
            pltpu.VMEM((8, nchp), jnp.float32),
        ],
        interpret=interpret,
    )(planes[0], planes[1], planes[2], qb)
    return nbr[:m], msk[:m] != 0


def _radius_knn(pos_all, pos_q, r, k, chunk=500):
    nq = pos_q.shape[0]
    qc = pos_q.reshape(nq // chunk, chunk, 3)

    def per_chunk(q):
        d = jnp.sum((q[:, None, :] - pos_all[None, :, :]) ** 2, axis=-1)
        within = d <= r * r
        neg = jnp.where(within, -d, -jnp.inf)
        vals, idxs = jax.lax.top_k(neg, k)
        return idxs.astype(jnp.int32), vals > -jnp.inf

    idxs, mask = jax.lax.map(per_chunk, qc)
    return idxs.reshape(nq, k), mask.reshape(nq, k)


def _ln_body(x_ref, g_ref, b_ref, o_ref):
    x = x_ref[...]
    mu = jnp.mean(x, axis=-1, keepdims=True)
    var = jnp.mean((x - mu) ** 2, axis=-1, keepdims=True)
    o_ref[...] = (x - mu) / jnp.sqrt(var + 1e-5) * g_ref[...] + b_ref[...]


def _layernorm(x, g, b):
    m, d = x.shape
    return pl.pallas_call(
        _ln_body,
        out_shape=jax.ShapeDtypeStruct((m, d), x.dtype),
    )(x, g.reshape(1, d), b.reshape(1, d))


def kernel(x, pos, batch, W_lin, b_lin, W_src, b_src, W_dst, b_dst,
           W_p1, b_p1, W_p2, b_p2, W_a, b_a, ln_g, ln_b):
    sel, pos_q = _fps_pallas(pos, _M)
    nbr, mask = _knn_pallas(pos, pos_q, _R * _R, _K)
    v = x @ W_lin + b_lin
    a_src = x @ W_src + b_src
    a_dst = x[sel] @ W_dst + b_dst
    xj = v[nbr]
    rel = pos_q[:, None, :] - pos[nbr]
    delta = jax.nn.relu(jax.nn.relu(rel @ W_p1 + b_p1) @ W_p2 + b_p2)
    alpha = a_dst[:, None, :] - a_src[nbr] + delta
    alpha = jax.nn.relu(alpha @ W_a + b_a)
    alpha = jnp.where(mask[:, :, None], alpha, -jnp.inf)
    alpha = jax.nn.softmax(alpha, axis=1)
    out = jnp.sum(alpha * (xj + delta), axis=1)
    out = _layernorm(out, ln_g, ln_b)
    return out, pos_q, batch[sel]


# speedup vs baseline: 3.3384x; 3.3384x over previous
import functools

import jax
import jax.numpy as jnp
from jax.experimental import pallas as pl
from jax.experimental.pallas import tpu as pltpu

_N = 50000
_D = 128
_M = 12500
_R = 0.1
_K = 16

_ROWS = 8
_IMAXV = 2**31 - 1


def _fps_body(m, rows, w, wp, planes_ref, lin_ref, pos0_ref, sel_ref,
              qx_ref, qy_ref, qz_ref, dists_ref):
    lin = lin_ref[...]
    dists_ref[...] = jnp.where(lin < jnp.int32(rows * w),
                               jnp.inf, -jnp.inf).astype(jnp.float32)
    lane128 = jax.lax.broadcasted_iota(jnp.int32, (1, 128), 1)
    lanew = jax.lax.broadcasted_iota(jnp.int32, (1, wp), 1)

    def body(i, carry):
        nxt, wx, wy, wz, a_s, a_x, a_y, a_z = carry
        laneq = lane128 == (i % 128)
        a_s = jnp.where(laneq, nxt, a_s)
        a_x = jnp.where(laneq, wx, a_x)
        a_y = jnp.where(laneq, wy, a_y)
        a_z = jnp.where(laneq, wz, a_z)
        blk = i // 128

        @pl.when((i % 128 == 127) | (i == m - 1))
        def _():
            sel_ref[pl.ds(blk, 1), :] = a_s
            qx_ref[pl.ds(blk, 1), :] = a_x
            qy_ref[pl.ds(blk, 1), :] = a_y
            qz_ref[pl.ds(blk, 1), :] = a_z

        dx = planes_ref[0] - wx
        dy = planes_ref[1] - wy
        dz = planes_ref[2] - wz
        d = (dx * dx + dz * dz) + dy * dy
        nd = jnp.minimum(dists_ref[...], d)
        dists_ref[...] = nd
        mx = jnp.max(nd)
        nxt2 = jnp.min(jnp.where(nd == mx, lin, jnp.int32(_IMAXV)))
        r = nxt2 // w
        c = nxt2 % w
        lmask = lanew == c
        wx2 = jnp.sum(jnp.where(lmask, planes_ref[0, pl.ds(r, 1), :], 0.0))
        wy2 = jnp.sum(jnp.where(lmask, planes_ref[1, pl.ds(r, 1), :], 0.0))
        wz2 = jnp.sum(jnp.where(lmask, planes_ref[2, pl.ds(r, 1), :], 0.0))
        return (nxt2, wx2, wy2, wz2, a_s, a_x, a_y, a_z)

    zf = jnp.zeros((1, 128), jnp.float32)
    zi = jnp.zeros((1, 128), jnp.int32)
    init = (jnp.int32(0), pos0_ref[0, 0], pos0_ref[0, 1], pos0_ref[0, 2],
            zi, zf, zf, zf)
    jax.lax.fori_loop(0, m, body, init)


def _fps_pallas(pos, m, interpret=False):
    n = pos.shape[0]
    rows = _ROWS
    w = n // rows
    wp = ((w + 127) // 128) * 128
    nb = (m + 127) // 128
    planes = pos.T.reshape(3, rows, w)
    if wp > w:
        planes = jnp.pad(planes, ((0, 0), (0, 0), (0, wp - w)))
    r_iota = jax.lax.broadcasted_iota(jnp.int32, (rows, wp), 0)
    c_iota = jax.lax.broadcasted_iota(jnp.int32, (rows, wp), 1)
    lin = jnp.where(c_iota < w, r_iota * w + c_iota, jnp.int32(_IMAXV))
    pos0 = pos[0:1, :]

    out = pl.pallas_call(
        functools.partial(_fps_body, m, rows, w, wp),
        grid=(1,),
        in_specs=[
            pl.BlockSpec((3, rows, wp), lambda i: (0, 0, 0)),
            pl.BlockSpec((rows, wp), lambda i: (0, 0)),
            pl.BlockSpec(memory_space=pltpu.SMEM),
        ],
        out_specs=[
            pl.BlockSpec((nb, 128), lambda i: (0, 0)),
            pl.BlockSpec((nb, 128), lambda i: (0, 0)),
            pl.BlockSpec((nb, 128), lambda i: (0, 0)),
            pl.BlockSpec((nb, 128), lambda i: (0, 0)),
        ],
        out_shape=[
            jax.ShapeDtypeStruct((nb, 128), jnp.int32),
            jax.ShapeDtypeStruct((nb, 128), jnp.float32),
            jax.ShapeDtypeStruct((nb, 128), jnp.float32),
            jax.ShapeDtypeStruct((nb, 128), jnp.float32),
        ],
        scratch_shapes=[pltpu.VMEM((rows, wp), jnp.float32)],
        interpret=interpret,
    )(planes, lin, pos0)
    sel = out[0].reshape(-1)[:m]
    pos_q = jnp.stack([o.reshape(-1)[:m] for o in out[1:]], axis=-1)
    return sel, pos_q


def _knn_body(nreal, nch, nchp, rr, k, px_ref, py_ref, pz_ref, q_ref,
              nbr_ref, msk_ref, neg_ref, mc_ref, cmp_ref, ntl_ref):
    qx = q_ref[0, 0]
    qy = q_ref[1, 0]
    qz = q_ref[2, 0]
    mc_ref[...] = jnp.full((8, nchp), -jnp.inf, jnp.float32)
    for j in range(nch):
        dx = px_ref[pl.ds(j, 1), :] - qx
        dy = py_ref[pl.ds(j, 1), :] - qy
        dz = pz_ref[pl.ds(j, 1), :] - qz
        d = (dx * dx + dz * dz) + dy * dy
        ng = jnp.where(d <= rr, -d, -jnp.inf)
        neg_ref[j] = ng
        mc_ref[:, j:j + 1] = jnp.max(ng, axis=1, keepdims=True)

    lane128 = jax.lax.broadcasted_iota(jnp.int32, (1, 128), 1)
    chunk2d = jax.lax.broadcasted_iota(jnp.int32, (8, nchp), 1)
    lane16 = jax.lax.broadcasted_iota(jnp.int32, (8, k), 1)

    mc = mc_ref[...]
    accc = jnp.zeros((8, k), jnp.int32)
    for t in range(k):
        m2 = jnp.max(mc, axis=1, keepdims=True)
        js = jnp.min(jnp.where(mc == m2, chunk2d, jnp.int32(_IMAXV)),
                     axis=1, keepdims=True)
        mc = jnp.where(chunk2d == js, -jnp.inf, mc)
        accc = jnp.where(lane16 == t, js, accc)

    for q in range(8):
        for t in range(k):
            c = accc[q, t]
            cmp_ref[q:q + 1, 128 * t:128 * (t + 1)] = \
                neg_ref[pl.ds(c, 1), q, :]
            ntl_ref[q:q + 1, 128 * t:128 * (t + 1)] = c * 128 + lane128

    vals = cmp_ref[...]
    ntile = ntl_ref[...]
    acc_nbr = jnp.zeros((8, k), jnp.int32)
    acc_msk = jnp.zeros((8, k), jnp.int32)
    for kk in range(k):
        m = jnp.max(vals, axis=1, keepdims=True)
        nsel = jnp.min(jnp.where(vals == m, ntile, jnp.int32(_IMAXV)),
                       axis=1, keepdims=True)
        vals = jnp.where(ntile == nsel, -jnp.inf, vals)
        acc_nbr = jnp.where(lane16 == kk, nsel, acc_nbr)
        acc_msk = jnp.where((lane16 == kk) & (m > -jnp.inf), 1, acc_msk)
    nbr_ref[...] = jnp.minimum(acc_nbr, jnp.int32(nreal - 1))
    msk_ref[...] = acc_msk


def _knn_pallas(pos, pos_q, rr, k, interpret=False):
    n = pos.shape[0]
    m = pos_q.shape[0]
    nch = (n + 127) // 128
    nchp = ((nch + 127) // 128) * 128
    nblk = (m + 7) // 8
    mp = nblk * 8
    planes = pos.T
    if nch * 128 > n:
        planes = jnp.pad(planes, ((0, 0), (0, nch * 128 - n)),
                         constant_values=1e9)
    planes = planes.reshape(3, nch, 128)
    qp = pos_q
    if mp > m:
        qp = jnp.pad(qp, ((0, mp - m), (0, 0)), constant_values=2.0)
    qb = qp.T.reshape(3, nblk, 8, 1)

    nbr, msk = pl.pallas_call(
        functools.partial(_knn_body, n, nch, nchp, rr, k),
        grid=(nblk,),
        in_specs=[
            pl.BlockSpec((nch, 128), lambda b: (0, 0)),
            pl.BlockSpec((nch, 128), lambda b: (0, 0)),
            pl.BlockSpec((nch, 128), lambda b: (0, 0)),
            pl.BlockSpec((3, 1, 8, 1), lambda b: (0, b, 0, 0)),
        ],
        out_specs=[
            pl.BlockSpec((8, k), lambda b: (b, 0)),
            pl.BlockSpec((8, k), lambda b: (b, 0)),
        ],
        out_shape=[
            jax.ShapeDtypeStruct((mp, k), jnp.int32),
            jax.ShapeDtypeStruct((mp, k), jnp.int32),
        ],
        scratch_shapes=[
            pltpu.VMEM((nch, 8, 128), jnp.float32),
            pltpu.VMEM((8, nchp), jnp.float32),
            pltpu.VMEM((8, k * 128), jnp.float32),
            pltpu.VMEM((8, k * 128), jnp.int32),
        ],
        interpret=interpret,
    )(planes[0], planes[1], planes[2], qb)
    return nbr[:m], msk[:m] != 0


def _radius_knn(pos_all, pos_q, r, k, chunk=500):
    nq = pos_q.shape[0]
    qc = pos_q.reshape(nq // chunk, chunk, 3)

    def per_chunk(q):
        d = jnp.sum((q[:, None, :] - pos_all[None, :, :]) ** 2, axis=-1)
        within = d <= r * r
        neg = jnp.where(within, -d, -jnp.inf)
        vals, idxs = jax.lax.top_k(neg, k)
        return idxs.astype(jnp.int32), vals > -jnp.inf

    idxs, mask = jax.lax.map(per_chunk, qc)
    return idxs.reshape(nq, k), mask.reshape(nq, k)


def _ln_body(x_ref, g_ref, b_ref, o_ref):
    x = x_ref[...]
    mu = jnp.mean(x, axis=-1, keepdims=True)
    var = jnp.mean((x - mu) ** 2, axis=-1, keepdims=True)
    o_ref[...] = (x - mu) / jnp.sqrt(var + 1e-5) * g_ref[...] + b_ref[...]


def _layernorm(x, g, b):
    m, d = x.shape
    return pl.pallas_call(
        _ln_body,
        out_shape=jax.ShapeDtypeStruct((m, d), x.dtype),
    )(x, g.reshape(1, d), b.reshape(1, d))


def kernel(x, pos, batch, W_lin, b_lin, W_src, b_src, W_dst, b_dst,
           W_p1, b_p1, W_p2, b_p2, W_a, b_a, ln_g, ln_b):
    sel, pos_q = _fps_pallas(pos, _M)
    nbr, mask = _knn_pallas(pos, pos_q, _R * _R, _K)
    v = x @ W_lin + b_lin
    a_src = x @ W_src + b_src
    a_dst = x[sel] @ W_dst + b_dst
    xj = v[nbr]
    rel = pos_q[:, None, :] - pos[nbr]
    delta = jax.nn.relu(jax.nn.relu(rel @ W_p1 + b_p1) @ W_p2 + b_p2)
    alpha = a_dst[:, None, :] - a_src[nbr] + delta
    alpha = jax.nn.relu(alpha @ W_a + b_a)
    alpha = jnp.where(mask[:, :, None], alpha, -jnp.inf)
    alpha = jax.nn.softmax(alpha, axis=1)
    out = jnp.sum(alpha * (xj + delta), axis=1)
    out = _layernorm(out, ln_g, ln_b)
    return out, pos_q, batch[sel]


# speedup vs baseline: 3.3644x; 1.0078x over previous
import functools

import jax
import jax.numpy as jnp
from jax import lax
from jax.experimental import pallas as pl
from jax.experimental.pallas import tpu as pltpu
from jax.experimental.pallas import tpu_sc as plsc

_N = 50000
_D = 128
_M = 12500
_R = 0.1
_K = 16

_ROWS = 8
_IMAXV = 2**31 - 1


def _fps_body(m, rows, w, wp, planes_ref, lin_ref, pos0_ref, sel_ref,
              qx_ref, qy_ref, qz_ref, dists_ref):
    lin = lin_ref[...]
    dists_ref[...] = jnp.where(lin < jnp.int32(rows * w),
                               jnp.inf, -jnp.inf).astype(jnp.float32)
    lane128 = jax.lax.broadcasted_iota(jnp.int32, (1, 128), 1)
    lanew = jax.lax.broadcasted_iota(jnp.int32, (1, wp), 1)

    def body(i, carry):
        nxt, wx, wy, wz, a_s, a_x, a_y, a_z = carry
        laneq = lane128 == (i % 128)
        a_s = jnp.where(laneq, nxt, a_s)
        a_x = jnp.where(laneq, wx, a_x)
        a_y = jnp.where(laneq, wy, a_y)
        a_z = jnp.where(laneq, wz, a_z)
        blk = i // 128

        @pl.when((i % 128 == 127) | (i == m - 1))
        def _():
            sel_ref[pl.ds(blk, 1), :] = a_s
            qx_ref[pl.ds(blk, 1), :] = a_x
            qy_ref[pl.ds(blk, 1), :] = a_y
            qz_ref[pl.ds(blk, 1), :] = a_z

        dx = planes_ref[0] - wx
        dy = planes_ref[1] - wy
        dz = planes_ref[2] - wz
        d = (dx * dx + dz * dz) + dy * dy
        nd = jnp.minimum(dists_ref[...], d)
        dists_ref[...] = nd
        mx = jnp.max(nd)
        nxt2 = jnp.min(jnp.where(nd == mx, lin, jnp.int32(_IMAXV)))
        r = nxt2 // w
        c = nxt2 % w
        lmask = lanew == c
        wx2 = jnp.sum(jnp.where(lmask, planes_ref[0, pl.ds(r, 1), :], 0.0))
        wy2 = jnp.sum(jnp.where(lmask, planes_ref[1, pl.ds(r, 1), :], 0.0))
        wz2 = jnp.sum(jnp.where(lmask, planes_ref[2, pl.ds(r, 1), :], 0.0))
        return (nxt2, wx2, wy2, wz2, a_s, a_x, a_y, a_z)

    zf = jnp.zeros((1, 128), jnp.float32)
    zi = jnp.zeros((1, 128), jnp.int32)
    init = (jnp.int32(0), pos0_ref[0, 0], pos0_ref[0, 1], pos0_ref[0, 2],
            zi, zf, zf, zf)
    jax.lax.fori_loop(0, m, body, init)


def _fps_pallas(pos, m, interpret=False):
    n = pos.shape[0]
    rows = _ROWS
    w = n // rows
    wp = ((w + 127) // 128) * 128
    nb = (m + 127) // 128
    planes = pos.T.reshape(3, rows, w)
    if wp > w:
        planes = jnp.pad(planes, ((0, 0), (0, 0), (0, wp - w)))
    r_iota = jax.lax.broadcasted_iota(jnp.int32, (rows, wp), 0)
    c_iota = jax.lax.broadcasted_iota(jnp.int32, (rows, wp), 1)
    lin = jnp.where(c_iota < w, r_iota * w + c_iota, jnp.int32(_IMAXV))
    pos0 = pos[0:1, :]

    out = pl.pallas_call(
        functools.partial(_fps_body, m, rows, w, wp),
        grid=(1,),
        in_specs=[
            pl.BlockSpec((3, rows, wp), lambda i: (0, 0, 0)),
            pl.BlockSpec((rows, wp), lambda i: (0, 0)),
            pl.BlockSpec(memory_space=pltpu.SMEM),
        ],
        out_specs=[
            pl.BlockSpec((nb, 128), lambda i: (0, 0)),
            pl.BlockSpec((nb, 128), lambda i: (0, 0)),
            pl.BlockSpec((nb, 128), lambda i: (0, 0)),
            pl.BlockSpec((nb, 128), lambda i: (0, 0)),
        ],
        out_shape=[
            jax.ShapeDtypeStruct((nb, 128), jnp.int32),
            jax.ShapeDtypeStruct((nb, 128), jnp.float32),
            jax.ShapeDtypeStruct((nb, 128), jnp.float32),
            jax.ShapeDtypeStruct((nb, 128), jnp.float32),
        ],
        scratch_shapes=[pltpu.VMEM((rows, wp), jnp.float32)],
        interpret=interpret,
    )(planes, lin, pos0)
    sel = out[0].reshape(-1)[:m]
    pos_q = jnp.stack([o.reshape(-1)[:m] for o in out[1:]], axis=-1)
    return sel, pos_q


def _knn_body(nreal, nch, nchp, rr, k, px_ref, py_ref, pz_ref, q_ref,
              nbr_ref, msk_ref, neg_ref, mc_ref, cmp_ref, ntl_ref):
    qx = q_ref[0, 0]
    qy = q_ref[1, 0]
    qz = q_ref[2, 0]
    mc_ref[...] = jnp.full((8, nchp), -jnp.inf, jnp.float32)
    for j in range(nch):
        dx = px_ref[pl.ds(j, 1), :] - qx
        dy = py_ref[pl.ds(j, 1), :] - qy
        dz = pz_ref[pl.ds(j, 1), :] - qz
        d = (dx * dx + dz * dz) + dy * dy
        ng = jnp.where(d <= rr, -d, -jnp.inf)
        neg_ref[j] = ng
        mc_ref[:, j:j + 1] = jnp.max(ng, axis=1, keepdims=True)

    lane128 = jax.lax.broadcasted_iota(jnp.int32, (1, 128), 1)
    chunk2d = jax.lax.broadcasted_iota(jnp.int32, (8, nchp), 1)
    lane16 = jax.lax.broadcasted_iota(jnp.int32, (8, k), 1)

    mc = mc_ref[...]
    accc = jnp.zeros((8, k), jnp.int32)
    for t in range(k):
        m2 = jnp.max(mc, axis=1, keepdims=True)
        js = jnp.min(jnp.where(mc == m2, chunk2d, jnp.int32(_IMAXV)),
                     axis=1, keepdims=True)
        mc = jnp.where(chunk2d == js, -jnp.inf, mc)
        accc = jnp.where(lane16 == t, js, accc)

    for q in range(8):
        for t in range(k):
            c = accc[q, t]
            cmp_ref[q:q + 1, 128 * t:128 * (t + 1)] = \
                neg_ref[pl.ds(c, 1), q, :]
            ntl_ref[q:q + 1, 128 * t:128 * (t + 1)] = c * 128 + lane128

    vals = cmp_ref[...]
    ntile = ntl_ref[...]
    acc_nbr = jnp.zeros((8, k), jnp.int32)
    acc_msk = jnp.zeros((8, k), jnp.int32)
    for kk in range(k):
        m = jnp.max(vals, axis=1, keepdims=True)
        nsel = jnp.min(jnp.where(vals == m, ntile, jnp.int32(_IMAXV)),
                       axis=1, keepdims=True)
        vals = jnp.where(ntile == nsel, -jnp.inf, vals)
        acc_nbr = jnp.where(lane16 == kk, nsel, acc_nbr)
        acc_msk = jnp.where((lane16 == kk) & (m > -jnp.inf), 1, acc_msk)
    nbr_ref[...] = jnp.minimum(acc_nbr, jnp.int32(nreal - 1))
    msk_ref[...] = acc_msk


def _knn_pallas(pos, pos_q, rr, k, interpret=False):
    n = pos.shape[0]
    m = pos_q.shape[0]
    nch = (n + 127) // 128
    nchp = ((nch + 127) // 128) * 128
    nblk = (m + 7) // 8
    mp = nblk * 8
    planes = pos.T
    if nch * 128 > n:
        planes = jnp.pad(planes, ((0, 0), (0, nch * 128 - n)),
                         constant_values=1e9)
    planes = planes.reshape(3, nch, 128)
    qp = pos_q
    if mp > m:
        qp = jnp.pad(qp, ((0, mp - m), (0, 0)), constant_values=2.0)
    qb = qp.T.reshape(3, nblk, 8, 1)

    nbr, msk = pl.pallas_call(
        functools.partial(_knn_body, n, nch, nchp, rr, k),
        grid=(nblk,),
        in_specs=[
            pl.BlockSpec((nch, 128), lambda b: (0, 0)),
            pl.BlockSpec((nch, 128), lambda b: (0, 0)),
            pl.BlockSpec((nch, 128), lambda b: (0, 0)),
            pl.BlockSpec((3, 1, 8, 1), lambda b: (0, b, 0, 0)),
        ],
        out_specs=[
            pl.BlockSpec((8, k), lambda b: (b, 0)),
            pl.BlockSpec((8, k), lambda b: (b, 0)),
        ],
        out_shape=[
            jax.ShapeDtypeStruct((mp, k), jnp.int32),
            jax.ShapeDtypeStruct((mp, k), jnp.int32),
        ],
        scratch_shapes=[
            pltpu.VMEM((nch, 8, 128), jnp.float32),
            pltpu.VMEM((8, nchp), jnp.float32),
            pltpu.VMEM((8, k * 128), jnp.float32),
            pltpu.VMEM((8, k * 128), jnp.int32),
        ],
        interpret=interpret,
    )(planes[0], planes[1], planes[2], qb)
    return nbr[:m], msk[:m]


def _mm3_body(x_ref, wl_ref, bl_ref, ws_ref, bs_ref, wd_ref, bd_ref,
              v_ref, as_ref, ad_ref):
    xx = x_ref[...]
    v_ref[...] = jnp.dot(xx, wl_ref[...],
                         preferred_element_type=jnp.float32) + bl_ref[...]
    as_ref[...] = jnp.dot(xx, ws_ref[...],
                          preferred_element_type=jnp.float32) + bs_ref[...]
    ad_ref[...] = jnp.dot(xx, wd_ref[...],
                          preferred_element_type=jnp.float32) + bd_ref[...]


def _mm3_pallas(x, wl, bl, ws, bs, wd, bd):
    n, d = x.shape
    blk = 1000
    g = n // blk
    shp = jax.ShapeDtypeStruct((n, d), jnp.float32)
    wspec = pl.BlockSpec((d, d), lambda i: (0, 0))
    bspec = pl.BlockSpec((1, d), lambda i: (0, 0))
    rspec = pl.BlockSpec((blk, d), lambda i: (i, 0))
    return pl.pallas_call(
        _mm3_body,
        grid=(g,),
        in_specs=[rspec, wspec, bspec, wspec, bspec, wspec, bspec],
        out_specs=[rspec, rspec, rspec],
        out_shape=[shp, shp, shp],
    )(x, wl, bl.reshape(1, d), ws, bs.reshape(1, d), wd, bd.reshape(1, d))


def _gather_sc(v, a_srcf, a_dstf, pos8, nbr_pad, sel_pad):
    ep = nbr_pad.shape[0]
    sp = sel_pad.shape[0]
    d = v.shape[1]
    nw = 32
    ch = 128
    nch_e = ep // (nw * ch)
    nch_s = sp // (nw * ch)
    mesh = plsc.VectorSubcoreMesh(core_axis_name="c", subcore_axis_name="s")

    @functools.partial(
        pl.kernel, mesh=mesh,
        out_type=[
            jax.ShapeDtypeStruct((ep, d), jnp.float32),
            jax.ShapeDtypeStruct((ep, d), jnp.float32),
            jax.ShapeDtypeStruct((ep, d), jnp.float32),
            jax.ShapeDtypeStruct((sp, d), jnp.float32),
        ],
        scratch_types=[
            pltpu.VMEM((ch,), jnp.int32),
            pltpu.VMEM((ch, d), jnp.float32),
            pltpu.SemaphoreType.DMA,
        ],
    )
    def k(v_hbm, asrc_hbm, adst_hbm, pos8_hbm, nbr_hbm, sel_hbm,
          oxj, oas, opos, oad, idx_v, rows_v, sem):
        wid = lax.axis_index("s") * 2 + lax.axis_index("c")

        def echunk(ci, _):
            off = wid * (nch_e * ch) + ci * ch
            pltpu.sync_copy(nbr_hbm.at[pl.ds(off, ch)], idx_v)
            pltpu.async_copy(v_hbm.at[idx_v], rows_v, sem).wait()
            pltpu.sync_copy(rows_v, oxj.at[pl.ds(off, ch)])
            pltpu.async_copy(asrc_hbm.at[idx_v], rows_v, sem).wait()
            pltpu.sync_copy(rows_v, oas.at[pl.ds(off, ch)])
            pltpu.async_copy(pos8_hbm.at[idx_v], rows_v, sem).wait()
            pltpu.sync_copy(rows_v, opos.at[pl.ds(off, ch)])
            return 0

        lax.fori_loop(0, nch_e, echunk, 0)

        def schunk(ci, _):
            off = wid * (nch_s * ch) + ci * ch
            pltpu.sync_copy(sel_hbm.at[pl.ds(off, ch)], idx_v)
            pltpu.async_copy(adst_hbm.at[idx_v], rows_v, sem).wait()
            pltpu.sync_copy(rows_v, oad.at[pl.ds(off, ch)])
            return 0

        lax.fori_loop(0, nch_s, schunk, 0)

    return k(v, a_srcf, a_dstf, pos8, nbr_pad, sel_pad)


def _edge_body(qb, k, xj_ref, as_ref, pj_ref, ad_ref, pq_ref, mk_ref,
               wp1_ref, bp1_ref, wp2_ref, bp2_ref, wa_ref, ba_ref,
               g_ref, b_ref, o_ref):
    e = qb * k
    d = 128
    rel = jnp.reshape(
        jnp.broadcast_to(jnp.reshape(pq_ref[...], (qb, 1, d)), (qb, k, d)),
        (e, d)) - pj_ref[...]
    h1 = jax.nn.relu(jnp.dot(rel, wp1_ref[...],
                             preferred_element_type=jnp.float32) + bp1_ref[...])
    delta = jax.nn.relu(jnp.dot(h1, wp2_ref[...],
                                preferred_element_type=jnp.float32) + bp2_ref[...])
    ad_e = jnp.reshape(
        jnp.broadcast_to(jnp.reshape(ad_ref[...], (qb, 1, d)), (qb, k, d)),
        (e, d))
    pre = ad_e - as_ref[...] + delta
    alpha = jax.nn.relu(jnp.dot(pre, wa_ref[...],
                                preferred_element_type=jnp.float32) + ba_ref[...])
    a3 = jnp.reshape(alpha, (qb, k, d))
    m3 = jnp.reshape(mk_ref[...], (qb, k, 1))
    a3 = jnp.where(m3 != 0, a3, -jnp.inf)
    mx = jnp.max(a3, axis=1, keepdims=True)
    ex = jnp.exp(a3 - mx)
    sm = ex / jnp.sum(ex, axis=1, keepdims=True)
    contrib = sm * jnp.reshape(xj_ref[...] + delta, (qb, k, d))
    out = jnp.sum(contrib, axis=1)
    mu = jnp.mean(out, axis=-1, keepdims=True)
    var = jnp.mean((out - mu) ** 2, axis=-1, keepdims=True)
    o_ref[...] = (out - mu) / jnp.sqrt(var + 1e-5) * g_ref[...] + b_ref[...]


def _edge_pallas(xj, asrc, posj8, adst, posq8, msk, wp1_8, bp1, wp2, bp2,
                 wa, ba, ln_g, ln_b, mq):
    d = 128
    k = _K
    qb = 128
    e = qb * k
    g = mq // qb
    espec = lambda w: pl.BlockSpec((e, w), lambda i: (i, 0))
    qspec = lambda w: pl.BlockSpec((qb, w), lambda i: (i, 0))
    cspec = lambda a, bdim: pl.BlockSpec((a, bdim), lambda i: (0, 0))
    return pl.pallas_call(
        functools.partial(_edge_body, qb, k),
        grid=(g,),
        in_specs=[
            espec(d), espec(d), espec(d), qspec(d), qspec(d), qspec(k),
            cspec(d, d), cspec(1, d), cspec(d, d), cspec(1, d),
            cspec(d, d), cspec(1, d), cspec(1, d), cspec(1, d),
        ],
        out_specs=qspec(d),
        out_shape=jax.ShapeDtypeStruct((mq, d), jnp.float32),
    )(xj, asrc, posj8, adst, posq8, msk, wp1_8, bp1.reshape(1, d), wp2,
      bp2.reshape(1, d), wa, ba.reshape(1, d), ln_g.reshape(1, d),
      ln_b.reshape(1, d))


def kernel(x, pos, batch, W_lin, b_lin, W_src, b_src, W_dst, b_dst,
           W_p1, b_p1, W_p2, b_p2, W_a, b_a, ln_g, ln_b):
    sel, pos_q = _fps_pallas(pos, _M)
    nbr, msk = _knn_pallas(pos, pos_q, _R * _R, _K)
    v, a_srcf, a_dstf = _mm3_pallas(x, W_lin, b_lin, W_src, b_src,
                                    W_dst, b_dst)
    pos8 = jnp.pad(pos, ((0, 0), (0, 125)))
    ep = 204800
    nbr_flat = jnp.pad(nbr.reshape(-1), (0, ep - _M * _K))
    sel_pad = jnp.pad(sel, (0, 16384 - _M))
    xj, asrc_e, posj8, adst = _gather_sc(v, a_srcf, a_dstf, pos8,
                                         nbr_flat, sel_pad)
    mq = 12800
    posq8 = jnp.pad(pos_q, ((0, mq - _M), (0, 125)))
    mskp = jnp.pad(msk, ((0, mq - _M), (0, 0)))
    wp1_8 = jnp.pad(W_p1, ((0, 125), (0, 0)))
    out = _edge_pallas(xj, asrc_e, posj8, adst, posq8, mskp, wp1_8, b_p1,
                       W_p2, b_p2, W_a, b_a, ln_g, ln_b, mq)[:_M]
    return out, pos_q, batch[sel]
